# baseline (speedup 1.0000x reference)
import functools

import jax
import jax.numpy as jnp
from jax import lax
from jax.experimental import pallas as pl
from jax.experimental.pallas import tpu as pltpu
from jax.experimental.pallas import tpu_sc as plsc

_N = 10000
_E = 320000
_D = 128
_G = 64
_NC = 2
_NS = 16
_NW = _NC * _NS
_EPW = _E // _NW
_C = 125
_NCH = _EPW // _C
_SS = 16
_NSS = _NCH // _SS
_NP = 10240
_RPT = _NP // _NS


def _deg_body(dstr_hbm, zeros1_hbm, out_hbm, dst_v, ones_v, bounce_v, dacc_sh):
    c = lax.axis_index("c")
    s = lax.axis_index("s")
    wid = s * _NC + c
    pltpu.sync_copy(dstr_hbm.at[wid], dst_v)
    for t in range(8):
        ones_v[pl.ds(16 * t, 16)] = jnp.ones((16,), jnp.float32)

    @pl.when(s == 0)
    def _():
        pltpu.sync_copy(zeros1_hbm, dacc_sh)

    plsc.subcore_barrier()

    @pl.loop(0, _NCH)
    def _(j):
        pltpu.sync_copy(ones_v.at[pl.ds(0, _C)], dacc_sh.at[dst_v.at[j]], add=True)

    plsc.subcore_barrier()

    @pl.when(s == 0)
    def _():
        pltpu.sync_copy(dacc_sh, bounce_v)
        pltpu.sync_copy(bounce_v, out_hbm.at[pl.ds(c * _N, _N)])


def _segsum_body(h_hbm, srcr_hbm, dstr_hbm, zeros2_hbm, out_hbm,
                 src_v, dst_v, buf_v, acc_sh, sem, ssem):
    c = lax.axis_index("c")
    s = lax.axis_index("s")
    wid = s * _NC + c
    pltpu.sync_copy(zeros2_hbm.at[pl.ds(s * _RPT, _RPT)],
                    acc_sh.at[pl.ds(s * _RPT, _RPT)])
    plsc.subcore_barrier()

    @pl.loop(0, _NSS)
    def _(g):
        pltpu.sync_copy(srcr_hbm.at[wid, pl.ds(g * _SS, _SS)], src_v)
        pltpu.sync_copy(dstr_hbm.at[wid, pl.ds(g * _SS, _SS)], dst_v)
        pltpu.async_copy(h_hbm.at[src_v.at[0]], buf_v.at[0], sem)

        @pl.loop(0, _SS)
        def _(j):
            par = lax.rem(j, 2)
            nxt = lax.rem(j + 1, 2)
            pltpu.make_async_copy(h_hbm.at[src_v.at[j]], buf_v.at[par],
                                  sem).wait()

            @pl.when(j >= 1)
            def _():
                pltpu.make_async_copy(buf_v.at[nxt],
                                      acc_sh.at[dst_v.at[j - 1]], ssem).wait()

            @pl.when(j + 1 < _SS)
            def _():
                pltpu.async_copy(h_hbm.at[src_v.at[j + 1]], buf_v.at[nxt], sem)

            pltpu.async_copy(buf_v.at[par], acc_sh.at[dst_v.at[j]], ssem,
                             add=True)

        pltpu.make_async_copy(buf_v.at[lax.rem(_SS - 1, 2)],
                              acc_sh.at[dst_v.at[_SS - 1]], ssem).wait()

    plsc.subcore_barrier()

    @pl.loop(0, _RPT // 64)
    def _(k):
        r0 = s * _RPT + k * 64
        pltpu.sync_copy(acc_sh.at[pl.ds(r0, 64)], buf_v.at[0, pl.ds(0, 64)])
        pltpu.sync_copy(buf_v.at[0, pl.ds(0, 64)], out_hbm.at[c, pl.ds(r0, 64)])


@functools.lru_cache(maxsize=None)
def _sc_kernels():
    mesh = plsc.VectorSubcoreMesh(core_axis_name="c", subcore_axis_name="s")
    deg_k = pl.kernel(
        _deg_body,
        out_type=jax.ShapeDtypeStruct((_NC * _N,), jnp.float32),
        mesh=mesh,
        scratch_types=[
            pltpu.VMEM((_NCH, _C), jnp.int32),
            pltpu.VMEM((128,), jnp.float32),
            pltpu.VMEM((_N,), jnp.float32),
            pltpu.VMEM_SHARED((_N,), jnp.float32),
        ],
    )
    segsum_k = pl.kernel(
        _segsum_body,
        out_type=jax.ShapeDtypeStruct((_NC, _NP, _D), jnp.bfloat16),
        mesh=mesh,
        scratch_types=[
            pltpu.VMEM((_SS, _C), jnp.int32),
            pltpu.VMEM((_SS, _C), jnp.int32),
            pltpu.VMEM((2, _C, _D), jnp.bfloat16),
            pltpu.VMEM_SHARED((_NP, _D), jnp.bfloat16),
            pltpu.SemaphoreType.DMA,
            pltpu.SemaphoreType.DMA,
        ],
        compiler_params=pltpu.CompilerParams(use_tc_tiling_on_sc=False),
    )
    return deg_k, segsum_k


def _deg(dstr, zeros1):
    return _sc_kernels()[0](dstr, zeros1)


def _segsum(h, srcr, dstr, zeros2):
    return _sc_kernels()[1](h, srcr, dstr, zeros2)


_BLK = 1000
_NBLK = _N // _BLK


def _dinv_of(pt_blk):
    deg = pt_blk[:, 0:1] + pt_blk[:, 1:2] + 1.0
    return lax.rsqrt(deg)


def _tc_xw_body(x_ref, w_ref, xw_ref):
    xw_ref[...] = jnp.dot(x_ref[...], w_ref[...],
                          preferred_element_type=jnp.float32)


def _tc_xw(x, w):
    return pl.pallas_call(
        _tc_xw_body,
        grid=(_NBLK,),
        in_specs=[
            pl.BlockSpec((_BLK, _D), lambda i: (i, 0)),
            pl.BlockSpec((_D, _D), lambda i: (0, 0)),
        ],
        out_specs=pl.BlockSpec((_BLK, _D), lambda i: (i, 0)),
        out_shape=jax.ShapeDtypeStruct((_N, _D), jnp.float32),
    )(x, w)


def _tc_scale_body(xw_ref, pt_ref, hb_ref):
    dinv = _dinv_of(pt_ref[...])
    hb_ref[...] = (xw_ref[...] * dinv).astype(jnp.bfloat16)


def _tc_scale(xw, pt):
    return pl.pallas_call(
        _tc_scale_body,
        grid=(_NBLK,),
        in_specs=[
            pl.BlockSpec((_BLK, _D), lambda i: (i, 0)),
            pl.BlockSpec((_BLK, 2), lambda i: (i, 0)),
        ],
        out_specs=pl.BlockSpec((_BLK, _D), lambda i: (i, 0)),
        out_shape=jax.ShapeDtypeStruct((_N, _D), jnp.bfloat16),
    )(xw, pt)


def _tc_mid_body(acc_ref, hp_ref, pt_ref, b_ref, w_ref, hb_ref):
    dinv = _dinv_of(pt_ref[...])
    tot = (acc_ref[0].astype(jnp.float32) + acc_ref[1].astype(jnp.float32)
           + hp_ref[...].astype(jnp.float32))
    xn = dinv * tot + b_ref[...][None, :]
    xn = jnp.maximum(xn, 0.0)
    res = jnp.dot(xn, w_ref[...], preferred_element_type=jnp.float32) * dinv
    hb_ref[...] = res.astype(jnp.bfloat16)


def _tc_mid(acc, hp, pt, b, w):
    return pl.pallas_call(
        _tc_mid_body,
        grid=(_NBLK,),
        in_specs=[
            pl.BlockSpec((_NC, _BLK, _D), lambda i: (0, i, 0)),
            pl.BlockSpec((_BLK, _D), lambda i: (i, 0)),
            pl.BlockSpec((_BLK, 2), lambda i: (i, 0)),
            pl.BlockSpec((_D,), lambda i: (0,)),
            pl.BlockSpec((_D, _D), lambda i: (0, 0)),
        ],
        out_specs=pl.BlockSpec((_BLK, _D), lambda i: (i, 0)),
        out_shape=jax.ShapeDtypeStruct((_N, _D), jnp.bfloat16),
    )(acc[:, :_N], hp, pt, b, w)


def _tc_final_body(acc_ref, hp_ref, pt_ref, b_ref, batch_ref,
                   wf1_ref, bf1_ref, wf2_ref, bf2_ref, out_ref,
                   sums_s, cnt_s):
    i = pl.program_id(0)

    @pl.when(i == 0)
    def _():
        sums_s[...] = jnp.zeros_like(sums_s)
        cnt_s[...] = jnp.zeros_like(cnt_s)

    dinv = _dinv_of(pt_ref[...])
    tot = (acc_ref[0].astype(jnp.float32) + acc_ref[1].astype(jnp.float32)
           + hp_ref[...].astype(jnp.float32))
    h3 = dinv * tot + b_ref[...][None, :]
    gids = lax.broadcasted_iota(jnp.int32, (_BLK, _G), 1)
    onehot = (batch_ref[...] == gids).astype(jnp.float32)
    sums_s[...] += lax.dot_general(onehot, h3, (((0,), (0,)), ((), ())),
                                   preferred_element_type=jnp.float32)
    cnt_s[...] += lax.dot_general(onehot, jnp.ones((_BLK, 1), jnp.float32),
                                  (((0,), (0,)), ((), ())),
                                  preferred_element_type=jnp.float32)

    @pl.when(i == _NBLK - 1)
    def _():
        pooled = sums_s[...] / jnp.maximum(cnt_s[...], 1.0)
        z = jnp.maximum(jnp.dot(pooled, wf1_ref[...],
                                preferred_element_type=jnp.float32)
                        + bf1_ref[...][None, :], 0.0)
        logit = jnp.dot(z, wf2_ref[...],
                        preferred_element_type=jnp.float32) + bf2_ref[...][None, :]
        out_ref[...] = jax.nn.sigmoid(logit)


def _tc_final(acc, hp, pt, b, batch2, wf1, bf1, wf2, bf2):
    return pl.pallas_call(
        _tc_final_body,
        grid=(_NBLK,),
        in_specs=[
            pl.BlockSpec((_NC, _BLK, _D), lambda i: (0, i, 0)),
            pl.BlockSpec((_BLK, _D), lambda i: (i, 0)),
            pl.BlockSpec((_BLK, 2), lambda i: (i, 0)),
            pl.BlockSpec((_D,), lambda i: (0,)),
            pl.BlockSpec((_BLK, 1), lambda i: (i, 0)),
            pl.BlockSpec((_D, _D), lambda i: (0, 0)),
            pl.BlockSpec((_D,), lambda i: (0,)),
            pl.BlockSpec((_D, 1), lambda i: (0, 0)),
            pl.BlockSpec((1,), lambda i: (0,)),
        ],
        out_specs=pl.BlockSpec((_G, 1), lambda i: (0, 0)),
        out_shape=jax.ShapeDtypeStruct((_G, 1), jnp.float32),
        scratch_shapes=[
            pltpu.VMEM((_G, _D), jnp.float32),
            pltpu.VMEM((_G, 1), jnp.float32),
        ],
    )(acc[:, :_N], hp, pt, b, batch2, wf1, bf1, wf2, bf2)


def kernel(x, edge_index, batch, W1, b1, W2, b2, W3, b3, Wf1, bf1, Wf2, bf2):
    srcr = edge_index[0].reshape(_NW, _NCH, _C)
    dstr = edge_index[1].reshape(_NW, _NCH, _C)
    zeros1 = jnp.zeros((_N,), jnp.float32)
    zeros2 = jnp.zeros((_NP, _D), jnp.bfloat16)

    xw1 = _tc_xw(x, W1)
    degp = _deg(dstr, zeros1).reshape(_NC, _N)
    pt = degp.T

    h1b = _tc_scale(xw1, pt)
    acc1 = _segsum(h1b, srcr, dstr, zeros2)
    h2b = _tc_mid(acc1, h1b, pt, b1, W2)
    acc2 = _segsum(h2b, srcr, dstr, zeros2)
    h3b = _tc_mid(acc2, h2b, pt, b2, W3)
    acc3 = _segsum(h3b, srcr, dstr, zeros2)
    return _tc_final(acc3, h3b, pt, b3, batch.reshape(_N, 1),
                     Wf1, bf1, Wf2, bf2)


# speedup vs baseline: 1.3924x; 1.3924x over previous
import functools

import jax
import jax.numpy as jnp
from jax import lax
from jax.experimental import pallas as pl
from jax.experimental.pallas import tpu as pltpu
from jax.experimental.pallas import tpu_sc as plsc

_N = 10000
_E = 320000
_D = 128
_G = 64
_NC = 2
_NS = 16
_NW = _NC * _NS
_EPW = _E // _NW
_C = 125
_NCH = _EPW // _C
_SS = 16
_NSS = _NCH // _SS
_NP = 10240
_RPT = _NP // _NS


def _deg_body(dstr_hbm, zeros1_hbm, out_hbm, dst_v, ones_v, bounce_v, dacc_sh):
    c = lax.axis_index("c")
    s = lax.axis_index("s")
    wid = s * _NC + c
    pltpu.sync_copy(dstr_hbm.at[wid], dst_v)
    for t in range(8):
        ones_v[pl.ds(16 * t, 16)] = jnp.ones((16,), jnp.float32)

    @pl.when(s == 0)
    def _():
        pltpu.sync_copy(zeros1_hbm, dacc_sh)

    plsc.subcore_barrier()

    @pl.loop(0, _NCH)
    def _(j):
        pltpu.sync_copy(ones_v.at[pl.ds(0, _C)], dacc_sh.at[dst_v.at[j]], add=True)

    plsc.subcore_barrier()

    @pl.when(s == 0)
    def _():
        pltpu.sync_copy(dacc_sh, bounce_v)
        pltpu.sync_copy(bounce_v, out_hbm.at[pl.ds(c * _N, _N)])


def _segsum_body(h_hbm, srcr_hbm, dstr_hbm, zeros2_hbm, out_hbm,
                 src_v, dst_v, buf_v, acc_sh, sem, ssem):
    c = lax.axis_index("c")
    s = lax.axis_index("s")
    wid = s * _NC + c
    pltpu.sync_copy(srcr_hbm.at[wid], src_v)
    pltpu.sync_copy(dstr_hbm.at[wid], dst_v)
    pltpu.sync_copy(zeros2_hbm.at[pl.ds(s * _RPT, _RPT)],
                    acc_sh.at[pl.ds(s * _RPT, _RPT)])
    plsc.subcore_barrier()

    def _gather_start(j, b):
        pltpu.async_copy(h_hbm.at[src_v.at[j]], buf_v.at[b], sem)

    def _gather_wait(j, b):
        pltpu.make_async_copy(h_hbm.at[src_v.at[j]], buf_v.at[b], sem).wait()

    def _scatter_start(j, b):
        pltpu.async_copy(buf_v.at[b], acc_sh.at[dst_v.at[j]], ssem, add=True)

    def _scatter_wait(j, b):
        pltpu.make_async_copy(buf_v.at[b], acc_sh.at[dst_v.at[j]],
                              ssem).wait()

    for b in range(3):
        _gather_start(b, b)

    @pl.loop(0, _NCH)
    def _(j):
        b = lax.rem(j, 4)
        _gather_wait(j, b)

        @pl.when(j >= 1)
        def _():
            _scatter_wait(j - 1, lax.rem(j - 1, 4))

        @pl.when(j + 3 < _NCH)
        def _():
            _gather_start(j + 3, lax.rem(j + 3, 4))

        _scatter_start(j, b)

    _scatter_wait(_NCH - 1, lax.rem(_NCH - 1, 4))
    plsc.subcore_barrier()

    @pl.loop(0, _RPT // 64)
    def _(k):
        r0 = s * _RPT + k * 64
        pltpu.sync_copy(acc_sh.at[pl.ds(r0, 64)], buf_v.at[0, pl.ds(0, 64)])
        pltpu.sync_copy(buf_v.at[0, pl.ds(0, 64)], out_hbm.at[c, pl.ds(r0, 64)])


@functools.lru_cache(maxsize=None)
def _sc_kernels():
    mesh = plsc.VectorSubcoreMesh(core_axis_name="c", subcore_axis_name="s")
    deg_k = pl.kernel(
        _deg_body,
        out_type=jax.ShapeDtypeStruct((_NC * _N,), jnp.float32),
        mesh=mesh,
        scratch_types=[
            pltpu.VMEM((_NCH, _C), jnp.int32),
            pltpu.VMEM((128,), jnp.float32),
            pltpu.VMEM((_N,), jnp.float32),
            pltpu.VMEM_SHARED((_N,), jnp.float32),
        ],
    )
    segsum_k = pl.kernel(
        _segsum_body,
        out_type=jax.ShapeDtypeStruct((_NC, _NP, _D), jnp.bfloat16),
        mesh=mesh,
        scratch_types=[
            pltpu.VMEM((_NCH, _C), jnp.int32),
            pltpu.VMEM((_NCH, _C), jnp.int32),
            pltpu.VMEM((4, _C, _D), jnp.bfloat16),
            pltpu.VMEM_SHARED((_NP, _D), jnp.bfloat16),
            pltpu.SemaphoreType.DMA,
            pltpu.SemaphoreType.DMA,
        ],
        compiler_params=pltpu.CompilerParams(use_tc_tiling_on_sc=False),
    )
    return deg_k, segsum_k


def _deg(dstr, zeros1):
    return _sc_kernels()[0](dstr, zeros1)


def _segsum(h, srcr, dstr, zeros2):
    return _sc_kernels()[1](h, srcr, dstr, zeros2)


_BLK = 1000
_NBLK = _N // _BLK


def _dinv_of(pt_blk):
    deg = pt_blk[:, 0:1] + pt_blk[:, 1:2] + 1.0
    return lax.rsqrt(deg)


def _tc_first_body(x_ref, w_ref, pt_ref, hb_ref):
    dinv = _dinv_of(pt_ref[...])
    res = jnp.dot(x_ref[...], w_ref[...],
                  preferred_element_type=jnp.float32) * dinv
    hb_ref[...] = res.astype(jnp.bfloat16)


def _tc_first(x, w, pt):
    return pl.pallas_call(
        _tc_first_body,
        grid=(_NBLK,),
        in_specs=[
            pl.BlockSpec((_BLK, _D), lambda i: (i, 0)),
            pl.BlockSpec((_D, _D), lambda i: (0, 0)),
            pl.BlockSpec((_BLK, 2), lambda i: (i, 0)),
        ],
        out_specs=pl.BlockSpec((_BLK, _D), lambda i: (i, 0)),
        out_shape=jax.ShapeDtypeStruct((_N, _D), jnp.bfloat16),
    )(x, w, pt)


def _tc_mid_body(acc_ref, hp_ref, pt_ref, b_ref, w_ref, hb_ref):
    dinv = _dinv_of(pt_ref[...])
    tot = (acc_ref[0].astype(jnp.float32) + acc_ref[1].astype(jnp.float32)
           + hp_ref[...].astype(jnp.float32))
    xn = dinv * tot + b_ref[...][None, :]
    xn = jnp.maximum(xn, 0.0)
    res = jnp.dot(xn, w_ref[...], preferred_element_type=jnp.float32) * dinv
    hb_ref[...] = res.astype(jnp.bfloat16)


def _tc_mid(acc, hp, pt, b, w):
    return pl.pallas_call(
        _tc_mid_body,
        grid=(_NBLK,),
        in_specs=[
            pl.BlockSpec((_NC, _BLK, _D), lambda i: (0, i, 0)),
            pl.BlockSpec((_BLK, _D), lambda i: (i, 0)),
            pl.BlockSpec((_BLK, 2), lambda i: (i, 0)),
            pl.BlockSpec((_D,), lambda i: (0,)),
            pl.BlockSpec((_D, _D), lambda i: (0, 0)),
        ],
        out_specs=pl.BlockSpec((_BLK, _D), lambda i: (i, 0)),
        out_shape=jax.ShapeDtypeStruct((_N, _D), jnp.bfloat16),
    )(acc[:, :_N], hp, pt, b, w)


def _tc_final_body(acc_ref, hp_ref, pt_ref, b_ref, batch_ref,
                   wf1_ref, bf1_ref, wf2_ref, bf2_ref, out_ref,
                   sums_s, cnt_s):
    i = pl.program_id(0)

    @pl.when(i == 0)
    def _():
        sums_s[...] = jnp.zeros_like(sums_s)
        cnt_s[...] = jnp.zeros_like(cnt_s)

    dinv = _dinv_of(pt_ref[...])
    tot = (acc_ref[0].astype(jnp.float32) + acc_ref[1].astype(jnp.float32)
           + hp_ref[...].astype(jnp.float32))
    h3 = dinv * tot + b_ref[...][None, :]
    gids = lax.broadcasted_iota(jnp.int32, (_BLK, _G), 1)
    onehot = (batch_ref[...] == gids).astype(jnp.float32)
    sums_s[...] += lax.dot_general(onehot, h3, (((0,), (0,)), ((), ())),
                                   preferred_element_type=jnp.float32)
    cnt_s[...] += lax.dot_general(onehot, jnp.ones((_BLK, 1), jnp.float32),
                                  (((0,), (0,)), ((), ())),
                                  preferred_element_type=jnp.float32)

    @pl.when(i == _NBLK - 1)
    def _():
        pooled = sums_s[...] / jnp.maximum(cnt_s[...], 1.0)
        z = jnp.maximum(jnp.dot(pooled, wf1_ref[...],
                                preferred_element_type=jnp.float32)
                        + bf1_ref[...][None, :], 0.0)
        logit = jnp.dot(z, wf2_ref[...],
                        preferred_element_type=jnp.float32) + bf2_ref[...][None, :]
        out_ref[...] = jax.nn.sigmoid(logit)


def _tc_final(acc, hp, pt, b, batch2, wf1, bf1, wf2, bf2):
    return pl.pallas_call(
        _tc_final_body,
        grid=(_NBLK,),
        in_specs=[
            pl.BlockSpec((_NC, _BLK, _D), lambda i: (0, i, 0)),
            pl.BlockSpec((_BLK, _D), lambda i: (i, 0)),
            pl.BlockSpec((_BLK, 2), lambda i: (i, 0)),
            pl.BlockSpec((_D,), lambda i: (0,)),
            pl.BlockSpec((_BLK, 1), lambda i: (i, 0)),
            pl.BlockSpec((_D, _D), lambda i: (0, 0)),
            pl.BlockSpec((_D,), lambda i: (0,)),
            pl.BlockSpec((_D, 1), lambda i: (0, 0)),
            pl.BlockSpec((1,), lambda i: (0,)),
        ],
        out_specs=pl.BlockSpec((_G, 1), lambda i: (0, 0)),
        out_shape=jax.ShapeDtypeStruct((_G, 1), jnp.float32),
        scratch_shapes=[
            pltpu.VMEM((_G, _D), jnp.float32),
            pltpu.VMEM((_G, 1), jnp.float32),
        ],
    )(acc[:, :_N], hp, pt, b, batch2, wf1, bf1, wf2, bf2)


def kernel(x, edge_index, batch, W1, b1, W2, b2, W3, b3, Wf1, bf1, Wf2, bf2):
    srcr = edge_index[0].reshape(_NW, _NCH, _C)
    dstr = edge_index[1].reshape(_NW, _NCH, _C)
    zeros1 = jnp.zeros((_N,), jnp.float32)
    zeros2 = jnp.zeros((_NP, _D), jnp.bfloat16)

    degp = _deg(dstr, zeros1).reshape(_NC, _N)
    pt = degp.T

    h1b = _tc_first(x, W1, pt)
    acc1 = _segsum(h1b, srcr, dstr, zeros2)
    h2b = _tc_mid(acc1, h1b, pt, b1, W2)
    acc2 = _segsum(h2b, srcr, dstr, zeros2)
    h3b = _tc_mid(acc2, h2b, pt, b2, W3)
    acc3 = _segsum(h3b, srcr, dstr, zeros2)
    return _tc_final(acc3, h3b, pt, b3, batch.reshape(_N, 1),
                     Wf1, bf1, Wf2, bf2)
